# baseline (speedup 1.0000x reference)
import functools

import jax
import jax.numpy as jnp
from jax import lax
from jax.experimental import pallas as pl
from jax.experimental.pallas import tpu as pltpu
from jax.experimental.pallas import tpu_sc as plsc

VOCAB = 1000000
D = 32
B = 16384
L = 200

VC = 64
VR = VOCAB // VC
VBM = 125
VGRID = VR // VBM

NC = 2
NS = 16
NW = NC * NS
TOTAL_IDX = B * L
SUB = 128
XR = TOTAL_IDX // SUB
ROWS_W = XR // NW
NSUB = 16
ITERS = ROWS_W // NSUB

PBB = 1024
PGRID = B // PBB


def _project_body(emb_ref, w_ref, v_ref):
    w = w_ref[0]
    v_ref[...] = jnp.sum(emb_ref[...] * w[None, None, :], axis=-1)[None]


@jax.jit
def _project(emb3, fc_w):
    return pl.pallas_call(
        _project_body,
        grid=(VGRID,),
        in_specs=[
            pl.BlockSpec((VBM, VC, D), lambda i: (i, 0, 0)),
            pl.BlockSpec((1, D), lambda i: (0, 0)),
        ],
        out_specs=pl.BlockSpec((1, VBM, VC), lambda i: (i, 0, 0)),
        out_shape=jax.ShapeDtypeStruct((VGRID, VBM, VC), jnp.float32),
    )(emb3, fc_w)


def _gather_body(v_hbm, x_hbm, out_hbm,
                 idx0, idx1, val0, val1, si0, si1, sg0, sg1, so0, so1):
    wid = lax.axis_index("s") * NC + lax.axis_index("c")
    base = wid * ROWS_W
    idxs, vals = (idx0, idx1), (val0, val1)
    sis, sgs, sos = (si0, si1), (sg0, sg1), (so0, so1)

    def idx_cp(i, b):
        return pltpu.make_async_copy(
            x_hbm.at[pl.ds(base + i * NSUB, NSUB)], idxs[b], sis[b])

    def gat_start(b):
        for j in range(NSUB):
            pltpu.make_async_copy(
                v_hbm.at[idxs[b].at[j]], vals[b].at[j], sgs[b]).start()

    def gat_wait(b):
        for j in range(NSUB):
            pltpu.make_async_copy(
                v_hbm.at[idxs[b].at[j]], vals[b].at[j], sgs[b]).wait()

    def out_cp(i, b):
        return pltpu.make_async_copy(
            vals[b], out_hbm.at[pl.ds(base + i * NSUB, NSUB)], sos[b])

    idx_cp(0, 0).start()

    def body(k, carry):
        for b in (0, 1):
            i = 2 * k + b
            o = 1 - b
            idx_cp(i, b).wait()

            @pl.when(i >= 2)
            def _():
                out_cp(i - 2, b).wait()

            gat_start(b)

            @pl.when(i >= 1)
            def _():
                gat_wait(o)
                out_cp(i - 1, o).start()

            @pl.when(i + 1 < ITERS)
            def _():
                idx_cp(i + 1, o).start()
        return carry

    lax.fori_loop(0, ITERS // 2, body, 0)
    last = ITERS - 1
    gat_wait(last % 2)
    out_cp(last, last % 2).start()
    out_cp(last - 1, (last - 1) % 2).wait()
    out_cp(last, last % 2).wait()


@jax.jit
def _gather(x2, v):
    mesh = plsc.VectorSubcoreMesh(
        core_axis_name="c", subcore_axis_name="s", num_cores=NC, num_subcores=NS
    )
    return pl.kernel(
        _gather_body,
        out_type=jax.ShapeDtypeStruct((XR, SUB), jnp.float32),
        mesh=mesh,
        scratch_types=[
            pltpu.VMEM((NSUB, SUB), jnp.int32),
            pltpu.VMEM((NSUB, SUB), jnp.int32),
            pltpu.VMEM((NSUB, SUB), jnp.float32),
            pltpu.VMEM((NSUB, SUB), jnp.float32),
            pltpu.SemaphoreType.DMA,
            pltpu.SemaphoreType.DMA,
            pltpu.SemaphoreType.DMA,
            pltpu.SemaphoreType.DMA,
            pltpu.SemaphoreType.DMA,
            pltpu.SemaphoreType.DMA,
        ],
    )(v, x2)


def _pool_body(g_ref, len_ref, b_ref, o_ref):
    pos = lax.broadcasted_iota(jnp.int32, (PBB, L), 1)
    lens = len_ref[...]
    masked = jnp.where(pos < lens, g_ref[...], 0.0)
    s = jnp.sum(masked, axis=1, keepdims=True)
    o_ref[...] = s / lens.astype(jnp.float32) + b_ref[0, 0]


@jax.jit
def _pool(g2, len2, fc_b2):
    return pl.pallas_call(
        _pool_body,
        grid=(PGRID,),
        in_specs=[
            pl.BlockSpec((PBB, L), lambda i: (i, 0)),
            pl.BlockSpec((PBB, 1), lambda i: (i, 0)),
            pl.BlockSpec((1, 1), lambda i: (0, 0)),
        ],
        out_specs=pl.BlockSpec((PBB, 1), lambda i: (i, 0)),
        out_shape=jax.ShapeDtypeStruct((B, 1), jnp.float32),
    )(g2, len2, fc_b2)


def kernel(x, lengths, emb_table, fc_w, fc_b):
    emb3 = emb_table.reshape(VR, VC, D)
    v = _project(emb3, fc_w).reshape(VOCAB)
    x2 = x.reshape(XR, SUB).astype(jnp.int32)
    g = _gather(x2, v)
    out = _pool(
        g.reshape(B, L),
        lengths.reshape(B, 1).astype(jnp.int32),
        fc_b.reshape(1, 1),
    )
    return out.reshape(B)


# speedup vs baseline: 1.0014x; 1.0014x over previous
import functools

import jax
import jax.numpy as jnp
from jax import lax
from jax.experimental import pallas as pl
from jax.experimental.pallas import tpu as pltpu
from jax.experimental.pallas import tpu_sc as plsc

VOCAB = 1000000
D = 32
B = 16384
L = 200

VC = 64
VR = VOCAB // VC
VBM = 125
VGRID = VR // VBM

NC = 2
NS = 16
NW = NC * NS
TOTAL_IDX = B * L
SUB = 128
XR = TOTAL_IDX // SUB
ROWS_W = XR // NW
NSUB = 40
ITERS = ROWS_W // NSUB

PBB = 1024
PGRID = B // PBB


def _project_body(emb_ref, w_ref, v_ref):
    w = w_ref[0]
    v_ref[...] = jnp.sum(emb_ref[...] * w[None, None, :], axis=-1)[None]


@jax.jit
def _project(emb3, fc_w):
    return pl.pallas_call(
        _project_body,
        grid=(VGRID,),
        in_specs=[
            pl.BlockSpec((VBM, VC, D), lambda i: (i, 0, 0)),
            pl.BlockSpec((1, D), lambda i: (0, 0)),
        ],
        out_specs=pl.BlockSpec((1, VBM, VC), lambda i: (i, 0, 0)),
        out_shape=jax.ShapeDtypeStruct((VGRID, VBM, VC), jnp.float32),
    )(emb3, fc_w)


def _gather_body(v_hbm, x_hbm, out_hbm,
                 idx0, idx1, val0, val1, si0, si1, sg0, sg1, so0, so1):
    wid = lax.axis_index("s") * NC + lax.axis_index("c")
    base = wid * ROWS_W
    idxs, vals = (idx0, idx1), (val0, val1)
    sis, sgs, sos = (si0, si1), (sg0, sg1), (so0, so1)

    def idx_cp(i, b):
        return pltpu.make_async_copy(
            x_hbm.at[pl.ds(base + i * NSUB, NSUB)], idxs[b], sis[b])

    def gat_start(b):
        for j in range(NSUB):
            pltpu.make_async_copy(
                v_hbm.at[idxs[b].at[j]], vals[b].at[j], sgs[b]).start()

    def gat_wait(b):
        for j in range(NSUB):
            pltpu.make_async_copy(
                v_hbm.at[idxs[b].at[j]], vals[b].at[j], sgs[b]).wait()

    def out_cp(i, b):
        return pltpu.make_async_copy(
            vals[b], out_hbm.at[pl.ds(base + i * NSUB, NSUB)], sos[b])

    idx_cp(0, 0).start()

    def body(k, carry):
        for b in (0, 1):
            i = 2 * k + b
            o = 1 - b
            idx_cp(i, b).wait()

            @pl.when(i >= 2)
            def _():
                out_cp(i - 2, b).wait()

            gat_start(b)

            @pl.when(i >= 1)
            def _():
                gat_wait(o)
                out_cp(i - 1, o).start()

            @pl.when(i + 1 < ITERS)
            def _():
                idx_cp(i + 1, o).start()
        return carry

    lax.fori_loop(0, ITERS // 2, body, 0)
    last = ITERS - 1
    gat_wait(last % 2)
    out_cp(last, last % 2).start()
    out_cp(last - 1, (last - 1) % 2).wait()
    out_cp(last, last % 2).wait()


@jax.jit
def _gather(x2, v):
    mesh = plsc.VectorSubcoreMesh(
        core_axis_name="c", subcore_axis_name="s", num_cores=NC, num_subcores=NS
    )
    return pl.kernel(
        _gather_body,
        out_type=jax.ShapeDtypeStruct((XR, SUB), jnp.float32),
        mesh=mesh,
        scratch_types=[
            pltpu.VMEM((NSUB, SUB), jnp.int32),
            pltpu.VMEM((NSUB, SUB), jnp.int32),
            pltpu.VMEM((NSUB, SUB), jnp.float32),
            pltpu.VMEM((NSUB, SUB), jnp.float32),
            pltpu.SemaphoreType.DMA,
            pltpu.SemaphoreType.DMA,
            pltpu.SemaphoreType.DMA,
            pltpu.SemaphoreType.DMA,
            pltpu.SemaphoreType.DMA,
            pltpu.SemaphoreType.DMA,
        ],
    )(v, x2)


def _pool_body(g_ref, len_ref, b_ref, o_ref):
    pos = lax.broadcasted_iota(jnp.int32, (PBB, L), 1)
    lens = len_ref[...]
    masked = jnp.where(pos < lens, g_ref[...], 0.0)
    s = jnp.sum(masked, axis=1, keepdims=True)
    o_ref[...] = s / lens.astype(jnp.float32) + b_ref[0, 0]


@jax.jit
def _pool(g2, len2, fc_b2):
    return pl.pallas_call(
        _pool_body,
        grid=(PGRID,),
        in_specs=[
            pl.BlockSpec((PBB, L), lambda i: (i, 0)),
            pl.BlockSpec((PBB, 1), lambda i: (i, 0)),
            pl.BlockSpec((1, 1), lambda i: (0, 0)),
        ],
        out_specs=pl.BlockSpec((PBB, 1), lambda i: (i, 0)),
        out_shape=jax.ShapeDtypeStruct((B, 1), jnp.float32),
    )(g2, len2, fc_b2)


def kernel(x, lengths, emb_table, fc_w, fc_b):
    emb3 = emb_table.reshape(VR, VC, D)
    v = _project(emb3, fc_w).reshape(VOCAB)
    x2 = x.reshape(XR, SUB).astype(jnp.int32)
    g = _gather(x2, v)
    out = _pool(
        g.reshape(B, L),
        lengths.reshape(B, 1).astype(jnp.int32),
        fc_b.reshape(1, 1),
    )
    return out.reshape(B)
